# Initial kernel scaffold; baseline (speedup 1.0000x reference)
import functools

import jax
import jax.numpy as jnp
from jax import lax
from jax.experimental import pallas as pl
from jax.experimental.pallas import tpu as pltpu
from jax.experimental.pallas import tpu_sc as plsc

N = 10000
D = 128
E_RAW = 320000 + N
NC = 2
NS = 16
CH = 128
K = 81
NW = NC * NS
E_PAD = NW * K * CH
NPAD = 10016
RPT = NPAD // NS
BM = 2000


def _sc_agg(hp, src3, dst3, zrows):
    mesh = plsc.VectorSubcoreMesh(core_axis_name="c", subcore_axis_name="s")

    @functools.partial(
        pl.kernel,
        out_type=jax.ShapeDtypeStruct((NC, NPAD, D), jnp.float32),
        mesh=mesh,
        scratch_types=[
            pltpu.VMEM((K, CH), jnp.int32),
            pltpu.VMEM((K, CH), jnp.int32),
            pltpu.VMEM((CH, D), jnp.float32),
            pltpu.VMEM_SHARED((NPAD, D), jnp.float32),
            pltpu.SemaphoreType.DMA,
        ],
    )
    def agg(hp_hbm, src_hbm, dst_hbm, z_hbm, out_hbm, src_v, dst_v, rows_v,
            acc_sh, sem):
        c = lax.axis_index("c")
        s = lax.axis_index("s")
        wid = c * NS + s
        pltpu.sync_copy(z_hbm.at[pl.ds(s * RPT, RPT)],
                        acc_sh.at[pl.ds(s * RPT, RPT)])
        pltpu.sync_copy(src_hbm.at[wid], src_v)
        pltpu.sync_copy(dst_hbm.at[wid], dst_v)
        plsc.subcore_barrier()

        def chunk(k, carry):
            pltpu.async_copy(hp_hbm.at[src_v.at[k]], rows_v, sem).wait()
            pltpu.sync_copy(rows_v, acc_sh.at[dst_v.at[k]], add=True)
            return carry

        lax.fori_loop(0, K, chunk, 0)
        plsc.subcore_barrier()
        pltpu.sync_copy(acc_sh.at[pl.ds(s * RPT, RPT)],
                        out_hbm.at[c, pl.ds(s * RPT, RPT)])

    return agg(hp, src3, dst3, zrows)


def _ln(xv, g, b):
    mu = jnp.mean(xv, axis=1, keepdims=True)
    xc = xv - mu
    var = jnp.mean(xc * xc, axis=1, keepdims=True)
    return xc * lax.rsqrt(var + 1e-5) * g + b


_AGG_SPEC = pl.BlockSpec((NC, BM, D), lambda i: (0, i, 0))
_ROW_SPEC = pl.BlockSpec((BM, D), lambda i: (i, 0))
_FULL_SPEC = pl.BlockSpec((D, D), lambda i: (0, 0))
_VEC_SPEC = pl.BlockSpec((1, D), lambda i: (0, 0))
_GRID = N // BM


def _tc_head(degacc, w0):
    def body(deg_ref, w_ref, dinv_ref, hp_ref):
        deg = deg_ref[0] + deg_ref[1]
        dinv = lax.rsqrt(jnp.maximum(deg, 1.0))
        dinv_ref[...] = dinv
        ones = jnp.ones((BM, D), jnp.float32)
        hp_ref[...] = dinv * jnp.dot(ones, w_ref[...],
                                     preferred_element_type=jnp.float32)

    return pl.pallas_call(
        body,
        grid=(_GRID,),
        in_specs=[_AGG_SPEC, _FULL_SPEC],
        out_specs=[_ROW_SPEC, _ROW_SPEC],
        out_shape=[jax.ShapeDtypeStruct((N, D), jnp.float32),
                   jax.ShapeDtypeStruct((N, D), jnp.float32)],
    )(degacc, w0)


def _tc_mid(aggout, dinvf, bias, wnext, z=None, add_one=False, ln=None,
            emit_x=False):
    ins = [aggout, dinvf, bias, wnext]
    specs = [_AGG_SPEC, _ROW_SPEC, _VEC_SPEC, _FULL_SPEC]
    if z is not None:
        ins.append(z)
        specs.append(_ROW_SPEC)
    if ln is not None:
        ins.extend(ln)
        specs.extend([_VEC_SPEC, _VEC_SPEC])

    def body(*refs):
        agg_ref, dinv_ref, b_ref, w_ref = refs[:4]
        pos = 4
        z_ref = None
        if z is not None:
            z_ref = refs[pos]
            pos += 1
        g_ref = bl_ref = None
        if ln is not None:
            g_ref, bl_ref = refs[pos], refs[pos + 1]
            pos += 2
        outs = refs[pos:]
        dinv = dinv_ref[...]
        xv = (agg_ref[0] + agg_ref[1]) * dinv + b_ref[...]
        if add_one:
            xv = xv + 1.0
        if z_ref is not None:
            xv = xv + z_ref[...]
        xv = jnp.maximum(xv, 0.0)
        if ln is not None:
            xv = _ln(xv, g_ref[...], bl_ref[...])
        outs[0][...] = dinv * jnp.dot(xv, w_ref[...],
                                      preferred_element_type=jnp.float32)
        if emit_x:
            outs[1][...] = xv

    n_out = 2 if emit_x else 1
    out = pl.pallas_call(
        body,
        grid=(_GRID,),
        in_specs=specs,
        out_specs=[_ROW_SPEC] * n_out,
        out_shape=[jax.ShapeDtypeStruct((N, D), jnp.float32)] * n_out,
    )(*ins)
    return out if emit_x else out[0]


def _tc_tail(aggout, dinvf, bias, z, g, b):
    def body(agg_ref, dinv_ref, b_ref, z_ref, g_ref, bl_ref, o_ref):
        xv = (agg_ref[0] + agg_ref[1]) * dinv_ref[...] + b_ref[...] + z_ref[...]
        o_ref[...] = _ln(xv, g_ref[...], bl_ref[...])

    return pl.pallas_call(
        body,
        grid=(_GRID,),
        in_specs=[_AGG_SPEC, _ROW_SPEC, _VEC_SPEC, _ROW_SPEC, _VEC_SPEC,
                  _VEC_SPEC],
        out_specs=_ROW_SPEC,
        out_shape=jax.ShapeDtypeStruct((N, D), jnp.float32),
    )(aggout, dinvf, bias, z, g, b)


@jax.jit
def kernel(x, edge_index, batch, Ws, bs, ln_hid_g, ln_hid_b, ln_out_g,
           ln_out_b):
    del x, batch
    loop = jnp.arange(N, dtype=jnp.int32)
    pad = E_PAD - E_RAW
    src = jnp.concatenate([edge_index[0].astype(jnp.int32), loop,
                           jnp.zeros((pad,), jnp.int32)])
    dst = jnp.concatenate([edge_index[1].astype(jnp.int32), loop,
                           jnp.full((pad,), N, jnp.int32)])
    src3 = src.reshape(NW, K, CH)
    dst3 = dst.reshape(NW, K, CH)
    zrows = jnp.zeros((NPAD, D), jnp.float32)
    ones_feat = jnp.ones((N, D), jnp.float32)

    b2d = bs.reshape(9, 1, D)
    g_h = ln_hid_g.reshape(1, D)
    b_h = ln_hid_b.reshape(1, D)
    g_o = ln_out_g.reshape(1, D)
    b_o = ln_out_b.reshape(1, D)

    degacc = _sc_agg(ones_feat, src3, dst3, zrows)
    dinvf, hp = _tc_head(degacc, Ws[0])

    z1 = z2 = None
    out = None
    for i in range(9):
        agg = _sc_agg(hp, src3, dst3, zrows)
        if i == 2:
            hp, z1 = _tc_mid(agg, dinvf, b2d[2], Ws[3], add_one=True,
                             ln=(g_h, b_h), emit_x=True)
        elif i == 5:
            hp, z2 = _tc_mid(agg, dinvf, b2d[5], Ws[6], z=z1,
                             ln=(g_h, b_h), emit_x=True)
        elif i == 8:
            out = _tc_tail(agg, dinvf, b2d[8], z2, g_o, b_o)
        else:
            hp = _tc_mid(agg, dinvf, b2d[i], Ws[i + 1])
    return out


# speedup vs baseline: 8.9321x; 8.9321x over previous
import functools

import jax
import jax.numpy as jnp
from jax import lax
from jax.experimental import pallas as pl
from jax.experimental.pallas import tpu as pltpu
from jax.experimental.pallas import tpu_sc as plsc

N = 10000
D = 128
E_RAW = 320000 + N
NC = 2
NS = 16
CH = 128
K = 81
NW = NC * NS
E_PAD = NW * K * CH
NPAD = 10112
RPT = NPAD // NS
BM = 2000


def _sc_agg(hp, src3, dst3, zrows):
    mesh = plsc.VectorSubcoreMesh(core_axis_name="c", subcore_axis_name="s")

    @functools.partial(
        pl.kernel,
        out_type=jax.ShapeDtypeStruct((NC, NPAD, D), jnp.float32),
        mesh=mesh,
        scratch_types=[
            pltpu.VMEM((K, CH), jnp.int32),
            pltpu.VMEM((K, CH), jnp.int32),
            pltpu.VMEM((CH, D), jnp.float32),
            pltpu.VMEM_SHARED((NPAD, D), jnp.float32),
            pltpu.SemaphoreType.DMA,
        ],
    )
    def agg(hp_hbm, src_hbm, dst_hbm, z_hbm, out_hbm, src_v, dst_v, rows_v,
            acc_sh, sem):
        c = lax.axis_index("c")
        s = lax.axis_index("s")
        wid = c * NS + s
        pltpu.sync_copy(z_hbm.at[pl.ds(s * RPT, RPT)],
                        acc_sh.at[pl.ds(s * RPT, RPT)])
        pltpu.sync_copy(src_hbm.at[wid], src_v)
        pltpu.sync_copy(dst_hbm.at[wid], dst_v)
        plsc.subcore_barrier()

        def chunk(k, carry):
            pltpu.async_copy(hp_hbm.at[src_v.at[k]], rows_v, sem).wait()
            pltpu.sync_copy(rows_v, acc_sh.at[dst_v.at[k]], add=True)
            return carry

        lax.fori_loop(0, K, chunk, 0)
        plsc.subcore_barrier()
        pltpu.sync_copy(acc_sh.at[pl.ds(s * RPT, RPT)],
                        out_hbm.at[c, pl.ds(s * RPT, RPT)])

    return agg(hp, src3, dst3, zrows)


def _ln(xv, g, b):
    mu = jnp.mean(xv, axis=1, keepdims=True)
    xc = xv - mu
    var = jnp.mean(xc * xc, axis=1, keepdims=True)
    return xc * lax.rsqrt(var + 1e-5) * g + b


_AGG_SPEC = pl.BlockSpec((NC, BM, D), lambda i: (0, i, 0))
_ROW_SPEC = pl.BlockSpec((BM, D), lambda i: (i, 0))
_FULL_SPEC = pl.BlockSpec((D, D), lambda i: (0, 0))
_VEC_SPEC = pl.BlockSpec((1, D), lambda i: (0, 0))
_GRID = N // BM


def _tc_head(degacc, w0):
    def body(deg_ref, w_ref, dinv_ref, hp_ref):
        deg = deg_ref[0] + deg_ref[1]
        dinv = lax.rsqrt(jnp.maximum(deg, 1.0))
        dinv_ref[...] = dinv
        ones = jnp.ones((BM, D), jnp.float32)
        hp_ref[...] = dinv * jnp.dot(ones, w_ref[...],
                                     preferred_element_type=jnp.float32)

    return pl.pallas_call(
        body,
        grid=(_GRID,),
        in_specs=[_AGG_SPEC, _FULL_SPEC],
        out_specs=[_ROW_SPEC, _ROW_SPEC],
        out_shape=[jax.ShapeDtypeStruct((N, D), jnp.float32),
                   jax.ShapeDtypeStruct((N, D), jnp.float32)],
    )(degacc, w0)


def _tc_mid(aggout, dinvf, bias, wnext, z=None, add_one=False, ln=None,
            emit_x=False):
    ins = [aggout, dinvf, bias, wnext]
    specs = [_AGG_SPEC, _ROW_SPEC, _VEC_SPEC, _FULL_SPEC]
    if z is not None:
        ins.append(z)
        specs.append(_ROW_SPEC)
    if ln is not None:
        ins.extend(ln)
        specs.extend([_VEC_SPEC, _VEC_SPEC])

    def body(*refs):
        agg_ref, dinv_ref, b_ref, w_ref = refs[:4]
        pos = 4
        z_ref = None
        if z is not None:
            z_ref = refs[pos]
            pos += 1
        g_ref = bl_ref = None
        if ln is not None:
            g_ref, bl_ref = refs[pos], refs[pos + 1]
            pos += 2
        outs = refs[pos:]
        dinv = dinv_ref[...]
        xv = (agg_ref[0] + agg_ref[1]) * dinv + b_ref[...]
        if add_one:
            xv = xv + 1.0
        if z_ref is not None:
            xv = xv + z_ref[...]
        xv = jnp.maximum(xv, 0.0)
        if ln is not None:
            xv = _ln(xv, g_ref[...], bl_ref[...])
        outs[0][...] = dinv * jnp.dot(xv, w_ref[...],
                                      preferred_element_type=jnp.float32)
        if emit_x:
            outs[1][...] = xv

    n_out = 2 if emit_x else 1
    out = pl.pallas_call(
        body,
        grid=(_GRID,),
        in_specs=specs,
        out_specs=[_ROW_SPEC] * n_out,
        out_shape=[jax.ShapeDtypeStruct((N, D), jnp.float32)] * n_out,
    )(*ins)
    return out if emit_x else out[0]


def _tc_tail(aggout, dinvf, bias, z, g, b):
    def body(agg_ref, dinv_ref, b_ref, z_ref, g_ref, bl_ref, o_ref):
        xv = (agg_ref[0] + agg_ref[1]) * dinv_ref[...] + b_ref[...] + z_ref[...]
        o_ref[...] = _ln(xv, g_ref[...], bl_ref[...])

    return pl.pallas_call(
        body,
        grid=(_GRID,),
        in_specs=[_AGG_SPEC, _ROW_SPEC, _VEC_SPEC, _ROW_SPEC, _VEC_SPEC,
                  _VEC_SPEC],
        out_specs=_ROW_SPEC,
        out_shape=jax.ShapeDtypeStruct((N, D), jnp.float32),
    )(aggout, dinvf, bias, z, g, b)


@jax.jit
def kernel(x, edge_index, batch, Ws, bs, ln_hid_g, ln_hid_b, ln_out_g,
           ln_out_b):
    del x, batch
    loop = jnp.arange(N, dtype=jnp.int32)
    pad = E_PAD - E_RAW
    src = jnp.concatenate([edge_index[0].astype(jnp.int32), loop,
                           jnp.zeros((pad,), jnp.int32)])
    dst = jnp.concatenate([edge_index[1].astype(jnp.int32), loop,
                           jnp.full((pad,), N, jnp.int32)])
    src3 = src.reshape(NW, K, CH)
    dst3 = dst.reshape(NW, K, CH)
    zrows = jnp.zeros((NPAD, D), jnp.float32)
    ones_feat = jnp.ones((N, D), jnp.float32)

    b2d = bs.reshape(9, 1, D)
    g_h = ln_hid_g.reshape(1, D)
    b_h = ln_hid_b.reshape(1, D)
    g_o = ln_out_g.reshape(1, D)
    b_o = ln_out_b.reshape(1, D)

    degacc = _sc_agg(ones_feat, src3, dst3, zrows)
    dinvf, hp = _tc_head(degacc, Ws[0])

    z1 = z2 = None
    out = None
    for i in range(9):
        agg = _sc_agg(hp, src3, dst3, zrows)
        if i == 2:
            hp, z1 = _tc_mid(agg, dinvf, b2d[2], Ws[3], add_one=True,
                             ln=(g_h, b_h), emit_x=True)
        elif i == 5:
            hp, z2 = _tc_mid(agg, dinvf, b2d[5], Ws[6], z=z1,
                             ln=(g_h, b_h), emit_x=True)
        elif i == 8:
            out = _tc_tail(agg, dinvf, b2d[8], z2, g_o, b_o)
        else:
            hp = _tc_mid(agg, dinvf, b2d[i], Ws[i + 1])
    return out


# speedup vs baseline: 11.9638x; 1.3394x over previous
import functools

import jax
import jax.numpy as jnp
from jax import lax
from jax.experimental import pallas as pl
from jax.experimental.pallas import tpu as pltpu
from jax.experimental.pallas import tpu_sc as plsc

N = 10000
D = 128
E_RAW = 320000 + N
NC = 2
NS = 16
CH = 128
K = 81
NW = NC * NS
E_PAD = NW * K * CH
NPAD = 10112
RPT = NPAD // NS
BM = 2000


NBUF = 3


def _sc_agg(hp, src3, dst3, zrows):
    mesh = plsc.VectorSubcoreMesh(core_axis_name="c", subcore_axis_name="s")

    @functools.partial(
        pl.kernel,
        out_type=jax.ShapeDtypeStruct((NC, NPAD, D), jnp.float32),
        mesh=mesh,
        scratch_types=[
            [pltpu.VMEM((CH,), jnp.int32) for _ in range(NBUF)],
            [pltpu.VMEM((CH,), jnp.int32) for _ in range(NBUF)],
            [pltpu.VMEM((CH, D), jnp.float32) for _ in range(NBUF)],
            pltpu.VMEM_SHARED((NPAD, D), jnp.float32),
            [pltpu.SemaphoreType.DMA for _ in range(NBUF)],
            [pltpu.SemaphoreType.DMA for _ in range(NBUF)],
        ],
    )
    def agg(hp_hbm, src_hbm, dst_hbm, z_hbm, out_hbm, sidx, didx, rows,
            acc_sh, isem, gsem):
        c = lax.axis_index("c")
        s = lax.axis_index("s")
        wid = c * NS + s
        pltpu.sync_copy(z_hbm.at[pl.ds(s * RPT, RPT)],
                        acc_sh.at[pl.ds(s * RPT, RPT)])
        for b in range(NBUF):
            pltpu.async_copy(src_hbm.at[wid, b], sidx[b], isem[b])
            pltpu.async_copy(dst_hbm.at[wid, b], didx[b], isem[b])
        plsc.subcore_barrier()
        for b in range(NBUF - 1):
            pltpu.make_async_copy(src_hbm.at[wid, b], sidx[b], isem[b]).wait()
            pltpu.make_async_copy(dst_hbm.at[wid, b], didx[b], isem[b]).wait()
            pltpu.async_copy(hp_hbm.at[sidx[b]], rows[b], gsem[b])

        def group(m, carry):
            base = m * NBUF
            for b in range(NBUF):
                k = base + b
                bp = (b + NBUF - 1) % NBUF
                @pl.when(k + NBUF - 1 < K)
                def _():
                    pltpu.make_async_copy(src_hbm.at[wid, 0], sidx[bp],
                                          isem[bp]).wait()
                    pltpu.make_async_copy(dst_hbm.at[wid, 0], didx[bp],
                                          isem[bp]).wait()
                    pltpu.async_copy(hp_hbm.at[sidx[bp]], rows[bp], gsem[bp])

                pltpu.make_async_copy(hp_hbm.at[sidx[b]], rows[b],
                                      gsem[b]).wait()
                pltpu.sync_copy(rows[b], acc_sh.at[didx[b]], add=True)

                @pl.when(k + NBUF < K)
                def _():
                    pltpu.async_copy(src_hbm.at[wid, k + NBUF], sidx[b],
                                     isem[b])
                    pltpu.async_copy(dst_hbm.at[wid, k + NBUF], didx[b],
                                     isem[b])
            return carry

        lax.fori_loop(0, K // NBUF, group, 0)
        plsc.subcore_barrier()
        pltpu.sync_copy(acc_sh.at[pl.ds(s * RPT, RPT)],
                        out_hbm.at[c, pl.ds(s * RPT, RPT)])

    return agg(hp, src3, dst3, zrows)


def _ln(xv, g, b):
    mu = jnp.mean(xv, axis=1, keepdims=True)
    xc = xv - mu
    var = jnp.mean(xc * xc, axis=1, keepdims=True)
    return xc * lax.rsqrt(var + 1e-5) * g + b


_AGG_SPEC = pl.BlockSpec((NC, BM, D), lambda i: (0, i, 0))
_ROW_SPEC = pl.BlockSpec((BM, D), lambda i: (i, 0))
_FULL_SPEC = pl.BlockSpec((D, D), lambda i: (0, 0))
_VEC_SPEC = pl.BlockSpec((1, D), lambda i: (0, 0))
_GRID = N // BM


def _tc_head(degacc, w0):
    def body(deg_ref, w_ref, dinv_ref, hp_ref):
        deg = deg_ref[0] + deg_ref[1]
        dinv = lax.rsqrt(jnp.maximum(deg, 1.0))
        dinv_ref[...] = dinv
        ones = jnp.ones((BM, D), jnp.float32)
        hp_ref[...] = dinv * jnp.dot(ones, w_ref[...],
                                     preferred_element_type=jnp.float32)

    return pl.pallas_call(
        body,
        grid=(_GRID,),
        in_specs=[_AGG_SPEC, _FULL_SPEC],
        out_specs=[_ROW_SPEC, _ROW_SPEC],
        out_shape=[jax.ShapeDtypeStruct((N, D), jnp.float32),
                   jax.ShapeDtypeStruct((N, D), jnp.float32)],
    )(degacc, w0)


def _tc_mid(aggout, dinvf, bias, wnext, z=None, add_one=False, ln=None,
            emit_x=False):
    ins = [aggout, dinvf, bias, wnext]
    specs = [_AGG_SPEC, _ROW_SPEC, _VEC_SPEC, _FULL_SPEC]
    if z is not None:
        ins.append(z)
        specs.append(_ROW_SPEC)
    if ln is not None:
        ins.extend(ln)
        specs.extend([_VEC_SPEC, _VEC_SPEC])

    def body(*refs):
        agg_ref, dinv_ref, b_ref, w_ref = refs[:4]
        pos = 4
        z_ref = None
        if z is not None:
            z_ref = refs[pos]
            pos += 1
        g_ref = bl_ref = None
        if ln is not None:
            g_ref, bl_ref = refs[pos], refs[pos + 1]
            pos += 2
        outs = refs[pos:]
        dinv = dinv_ref[...]
        xv = (agg_ref[0] + agg_ref[1]) * dinv + b_ref[...]
        if add_one:
            xv = xv + 1.0
        if z_ref is not None:
            xv = xv + z_ref[...]
        xv = jnp.maximum(xv, 0.0)
        if ln is not None:
            xv = _ln(xv, g_ref[...], bl_ref[...])
        outs[0][...] = dinv * jnp.dot(xv, w_ref[...],
                                      preferred_element_type=jnp.float32)
        if emit_x:
            outs[1][...] = xv

    n_out = 2 if emit_x else 1
    out = pl.pallas_call(
        body,
        grid=(_GRID,),
        in_specs=specs,
        out_specs=[_ROW_SPEC] * n_out,
        out_shape=[jax.ShapeDtypeStruct((N, D), jnp.float32)] * n_out,
    )(*ins)
    return out if emit_x else out[0]


def _tc_tail(aggout, dinvf, bias, z, g, b):
    def body(agg_ref, dinv_ref, b_ref, z_ref, g_ref, bl_ref, o_ref):
        xv = (agg_ref[0] + agg_ref[1]) * dinv_ref[...] + b_ref[...] + z_ref[...]
        o_ref[...] = _ln(xv, g_ref[...], bl_ref[...])

    return pl.pallas_call(
        body,
        grid=(_GRID,),
        in_specs=[_AGG_SPEC, _ROW_SPEC, _VEC_SPEC, _ROW_SPEC, _VEC_SPEC,
                  _VEC_SPEC],
        out_specs=_ROW_SPEC,
        out_shape=jax.ShapeDtypeStruct((N, D), jnp.float32),
    )(aggout, dinvf, bias, z, g, b)


@jax.jit
def kernel(x, edge_index, batch, Ws, bs, ln_hid_g, ln_hid_b, ln_out_g,
           ln_out_b):
    del x, batch
    loop = jnp.arange(N, dtype=jnp.int32)
    pad = E_PAD - E_RAW
    src = jnp.concatenate([edge_index[0].astype(jnp.int32), loop,
                           jnp.zeros((pad,), jnp.int32)])
    dst = jnp.concatenate([edge_index[1].astype(jnp.int32), loop,
                           jnp.full((pad,), N, jnp.int32)])
    src3 = src.reshape(NW, K, CH)
    dst3 = dst.reshape(NW, K, CH)
    zrows = jnp.zeros((NPAD, D), jnp.float32)
    ones_feat = jnp.ones((N, D), jnp.float32)

    b2d = bs.reshape(9, 1, D)
    g_h = ln_hid_g.reshape(1, D)
    b_h = ln_hid_b.reshape(1, D)
    g_o = ln_out_g.reshape(1, D)
    b_o = ln_out_b.reshape(1, D)

    degacc = _sc_agg(ones_feat, src3, dst3, zrows)
    dinvf, hp = _tc_head(degacc, Ws[0])

    z1 = z2 = None
    out = None
    for i in range(9):
        agg = _sc_agg(hp, src3, dst3, zrows)
        if i == 2:
            hp, z1 = _tc_mid(agg, dinvf, b2d[2], Ws[3], add_one=True,
                             ln=(g_h, b_h), emit_x=True)
        elif i == 5:
            hp, z2 = _tc_mid(agg, dinvf, b2d[5], Ws[6], z=z1,
                             ln=(g_h, b_h), emit_x=True)
        elif i == 8:
            out = _tc_tail(agg, dinvf, b2d[8], z2, g_o, b_o)
        else:
            hp = _tc_mid(agg, dinvf, b2d[i], Ws[i + 1])
    return out


# speedup vs baseline: 12.5641x; 1.0502x over previous
import functools

import jax
import jax.numpy as jnp
from jax import lax
from jax.experimental import pallas as pl
from jax.experimental.pallas import tpu as pltpu
from jax.experimental.pallas import tpu_sc as plsc

N = 10000
D = 128
E_RAW = 320000 + N
NC = 2
NS = 16
CH = 96
K = 108
NW = NC * NS
E_PAD = NW * K * CH
NPAD = 10112
RPT = NPAD // NS
BM = 2000


NBUF = 4


def _sc_agg(hp, src3, dst3, zrows):
    mesh = plsc.VectorSubcoreMesh(core_axis_name="c", subcore_axis_name="s")

    @functools.partial(
        pl.kernel,
        out_type=jax.ShapeDtypeStruct((NC, NPAD, D), jnp.float32),
        mesh=mesh,
        scratch_types=[
            [pltpu.VMEM((CH,), jnp.int32) for _ in range(NBUF)],
            [pltpu.VMEM((CH,), jnp.int32) for _ in range(NBUF)],
            [pltpu.VMEM((CH, D), jnp.float32) for _ in range(NBUF)],
            pltpu.VMEM_SHARED((NPAD, D), jnp.float32),
            [pltpu.SemaphoreType.DMA for _ in range(NBUF)],
            [pltpu.SemaphoreType.DMA for _ in range(NBUF)],
            [pltpu.SemaphoreType.DMA for _ in range(NBUF)],
        ],
    )
    def agg(hp_hbm, src_hbm, dst_hbm, z_hbm, out_hbm, sidx, didx, rows,
            acc_sh, isem, gsem, ssem):
        c = lax.axis_index("c")
        s = lax.axis_index("s")
        wid = c * NS + s
        pltpu.sync_copy(z_hbm.at[pl.ds(s * RPT, RPT)],
                        acc_sh.at[pl.ds(s * RPT, RPT)])
        for b in range(3):
            pltpu.async_copy(src_hbm.at[wid, b], sidx[b], isem[b])
            pltpu.async_copy(dst_hbm.at[wid, b], didx[b], isem[b])
        plsc.subcore_barrier()
        for b in range(2):
            pltpu.make_async_copy(src_hbm.at[wid, b], sidx[b], isem[b]).wait()
            pltpu.make_async_copy(dst_hbm.at[wid, b], didx[b], isem[b]).wait()
            pltpu.async_copy(hp_hbm.at[sidx[b]], rows[b], gsem[b])

        def group(m, carry):
            base = m * NBUF
            for b in range(NBUF):
                k = base + b
                b3 = (b + 3) % NBUF
                b2 = (b + 2) % NBUF

                @pl.when(k >= 1)
                def _():
                    pltpu.make_async_copy(rows[b3], acc_sh.at[didx[b3]],
                                          ssem[b3]).wait()

                @pl.when(k + 3 < K)
                def _():
                    pltpu.async_copy(src_hbm.at[wid, k + 3], sidx[b3],
                                     isem[b3])
                    pltpu.async_copy(dst_hbm.at[wid, k + 3], didx[b3],
                                     isem[b3])

                @pl.when(k + 2 < K)
                def _():
                    pltpu.make_async_copy(src_hbm.at[wid, 0], sidx[b2],
                                          isem[b2]).wait()
                    pltpu.make_async_copy(dst_hbm.at[wid, 0], didx[b2],
                                          isem[b2]).wait()
                    pltpu.async_copy(hp_hbm.at[sidx[b2]], rows[b2], gsem[b2])

                pltpu.make_async_copy(hp_hbm.at[sidx[b]], rows[b],
                                      gsem[b]).wait()
                pltpu.async_copy(rows[b], acc_sh.at[didx[b]], ssem[b],
                                 add=True)
            return carry

        lax.fori_loop(0, K // NBUF, group, 0)
        bl = (K - 1) % NBUF
        pltpu.make_async_copy(rows[bl], acc_sh.at[didx[bl]], ssem[bl]).wait()
        plsc.subcore_barrier()
        pltpu.sync_copy(acc_sh.at[pl.ds(s * RPT, RPT)],
                        out_hbm.at[c, pl.ds(s * RPT, RPT)])

    return agg(hp, src3, dst3, zrows)


def _ln(xv, g, b):
    mu = jnp.mean(xv, axis=1, keepdims=True)
    xc = xv - mu
    var = jnp.mean(xc * xc, axis=1, keepdims=True)
    return xc * lax.rsqrt(var + 1e-5) * g + b


_AGG_SPEC = pl.BlockSpec((NC, BM, D), lambda i: (0, i, 0))
_ROW_SPEC = pl.BlockSpec((BM, D), lambda i: (i, 0))
_FULL_SPEC = pl.BlockSpec((D, D), lambda i: (0, 0))
_VEC_SPEC = pl.BlockSpec((1, D), lambda i: (0, 0))
_GRID = N // BM


def _tc_head(degacc, w0):
    def body(deg_ref, w_ref, dinv_ref, hp_ref):
        deg = deg_ref[0] + deg_ref[1]
        dinv = lax.rsqrt(jnp.maximum(deg, 1.0))
        dinv_ref[...] = dinv
        ones = jnp.ones((BM, D), jnp.float32)
        hp_ref[...] = dinv * jnp.dot(ones, w_ref[...],
                                     preferred_element_type=jnp.float32)

    return pl.pallas_call(
        body,
        grid=(_GRID,),
        in_specs=[_AGG_SPEC, _FULL_SPEC],
        out_specs=[_ROW_SPEC, _ROW_SPEC],
        out_shape=[jax.ShapeDtypeStruct((N, D), jnp.float32),
                   jax.ShapeDtypeStruct((N, D), jnp.float32)],
    )(degacc, w0)


def _tc_mid(aggout, dinvf, bias, wnext, z=None, add_one=False, ln=None,
            emit_x=False):
    ins = [aggout, dinvf, bias, wnext]
    specs = [_AGG_SPEC, _ROW_SPEC, _VEC_SPEC, _FULL_SPEC]
    if z is not None:
        ins.append(z)
        specs.append(_ROW_SPEC)
    if ln is not None:
        ins.extend(ln)
        specs.extend([_VEC_SPEC, _VEC_SPEC])

    def body(*refs):
        agg_ref, dinv_ref, b_ref, w_ref = refs[:4]
        pos = 4
        z_ref = None
        if z is not None:
            z_ref = refs[pos]
            pos += 1
        g_ref = bl_ref = None
        if ln is not None:
            g_ref, bl_ref = refs[pos], refs[pos + 1]
            pos += 2
        outs = refs[pos:]
        dinv = dinv_ref[...]
        xv = (agg_ref[0] + agg_ref[1]) * dinv + b_ref[...]
        if add_one:
            xv = xv + 1.0
        if z_ref is not None:
            xv = xv + z_ref[...]
        xv = jnp.maximum(xv, 0.0)
        if ln is not None:
            xv = _ln(xv, g_ref[...], bl_ref[...])
        outs[0][...] = dinv * jnp.dot(xv, w_ref[...],
                                      preferred_element_type=jnp.float32)
        if emit_x:
            outs[1][...] = xv

    n_out = 2 if emit_x else 1
    out = pl.pallas_call(
        body,
        grid=(_GRID,),
        in_specs=specs,
        out_specs=[_ROW_SPEC] * n_out,
        out_shape=[jax.ShapeDtypeStruct((N, D), jnp.float32)] * n_out,
    )(*ins)
    return out if emit_x else out[0]


def _tc_tail(aggout, dinvf, bias, z, g, b):
    def body(agg_ref, dinv_ref, b_ref, z_ref, g_ref, bl_ref, o_ref):
        xv = (agg_ref[0] + agg_ref[1]) * dinv_ref[...] + b_ref[...] + z_ref[...]
        o_ref[...] = _ln(xv, g_ref[...], bl_ref[...])

    return pl.pallas_call(
        body,
        grid=(_GRID,),
        in_specs=[_AGG_SPEC, _ROW_SPEC, _VEC_SPEC, _ROW_SPEC, _VEC_SPEC,
                  _VEC_SPEC],
        out_specs=_ROW_SPEC,
        out_shape=jax.ShapeDtypeStruct((N, D), jnp.float32),
    )(aggout, dinvf, bias, z, g, b)


@jax.jit
def kernel(x, edge_index, batch, Ws, bs, ln_hid_g, ln_hid_b, ln_out_g,
           ln_out_b):
    del x, batch
    loop = jnp.arange(N, dtype=jnp.int32)
    pad = E_PAD - E_RAW
    src = jnp.concatenate([edge_index[0].astype(jnp.int32), loop,
                           jnp.zeros((pad,), jnp.int32)])
    dst = jnp.concatenate([edge_index[1].astype(jnp.int32), loop,
                           jnp.full((pad,), N, jnp.int32)])
    src3 = src.reshape(NW, K, CH)
    dst3 = dst.reshape(NW, K, CH)
    zrows = jnp.zeros((NPAD, D), jnp.float32)
    ones_feat = jnp.ones((N, D), jnp.float32)

    b2d = bs.reshape(9, 1, D)
    g_h = ln_hid_g.reshape(1, D)
    b_h = ln_hid_b.reshape(1, D)
    g_o = ln_out_g.reshape(1, D)
    b_o = ln_out_b.reshape(1, D)

    degacc = _sc_agg(ones_feat, src3, dst3, zrows)
    dinvf, hp = _tc_head(degacc, Ws[0])

    z1 = z2 = None
    out = None
    for i in range(9):
        agg = _sc_agg(hp, src3, dst3, zrows)
        if i == 2:
            hp, z1 = _tc_mid(agg, dinvf, b2d[2], Ws[3], add_one=True,
                             ln=(g_h, b_h), emit_x=True)
        elif i == 5:
            hp, z2 = _tc_mid(agg, dinvf, b2d[5], Ws[6], z=z1,
                             ln=(g_h, b_h), emit_x=True)
        elif i == 8:
            out = _tc_tail(agg, dinvf, b2d[8], z2, g_o, b_o)
        else:
            hp = _tc_mid(agg, dinvf, b2d[i], Ws[i + 1])
    return out


# speedup vs baseline: 14.7932x; 1.1774x over previous
import functools

import jax
import jax.numpy as jnp
from jax import lax
from jax.experimental import pallas as pl
from jax.experimental.pallas import tpu as pltpu
from jax.experimental.pallas import tpu_sc as plsc

N = 10000
D = 128
E_RAW = 320000 + N
NC = 2
NS = 16
CH = 96
K = 108
NW = NC * NS
E_PAD = NW * K * CH
NPAD = 10112
RPT = NPAD // NS
BM = 2000


NBUF = 4


def _make_sc_agg(W):
    mesh = plsc.VectorSubcoreMesh(core_axis_name="c", subcore_axis_name="s")

    @functools.partial(
        pl.kernel,
        out_type=jax.ShapeDtypeStruct((NC, NPAD, W), jnp.float32),
        mesh=mesh,
        scratch_types=[
            [pltpu.VMEM((CH,), jnp.int32) for _ in range(NBUF)],
            [pltpu.VMEM((CH,), jnp.int32) for _ in range(NBUF)],
            [pltpu.VMEM((CH, W), jnp.float32) for _ in range(NBUF)],
            pltpu.VMEM_SHARED((NPAD, W), jnp.float32),
            [pltpu.SemaphoreType.DMA for _ in range(NBUF)],
            [pltpu.SemaphoreType.DMA for _ in range(NBUF)],
            [pltpu.SemaphoreType.DMA for _ in range(NBUF)],
        ],
        compiler_params=pltpu.CompilerParams(use_tc_tiling_on_sc=False),
    )
    def agg(hp_hbm, src_hbm, dst_hbm, z_hbm, out_hbm, sidx, didx, rows,
            acc_sh, isem, gsem, ssem):
        c = lax.axis_index("c")
        s = lax.axis_index("s")
        wid = c * NS + s
        pltpu.sync_copy(z_hbm.at[pl.ds(s * RPT, RPT)],
                        acc_sh.at[pl.ds(s * RPT, RPT)])
        for b in range(3):
            pltpu.async_copy(src_hbm.at[wid, b], sidx[b], isem[b])
            pltpu.async_copy(dst_hbm.at[wid, b], didx[b], isem[b])
        plsc.subcore_barrier()
        for b in range(2):
            pltpu.make_async_copy(src_hbm.at[wid, b], sidx[b], isem[b]).wait()
            pltpu.make_async_copy(dst_hbm.at[wid, b], didx[b], isem[b]).wait()
            pltpu.async_copy(hp_hbm.at[sidx[b]], rows[b], gsem[b])

        def group(m, carry):
            base = m * NBUF
            for b in range(NBUF):
                k = base + b
                b3 = (b + 3) % NBUF
                b2 = (b + 2) % NBUF

                @pl.when(k >= 1)
                def _():
                    pltpu.make_async_copy(rows[b3], acc_sh.at[didx[b3]],
                                          ssem[b3]).wait()

                @pl.when(k + 3 < K)
                def _():
                    pltpu.async_copy(src_hbm.at[wid, k + 3], sidx[b3],
                                     isem[b3])
                    pltpu.async_copy(dst_hbm.at[wid, k + 3], didx[b3],
                                     isem[b3])

                @pl.when(k + 2 < K)
                def _():
                    pltpu.make_async_copy(src_hbm.at[wid, 0], sidx[b2],
                                          isem[b2]).wait()
                    pltpu.make_async_copy(dst_hbm.at[wid, 0], didx[b2],
                                          isem[b2]).wait()
                    pltpu.async_copy(hp_hbm.at[sidx[b2]], rows[b2], gsem[b2])

                pltpu.make_async_copy(hp_hbm.at[sidx[b]], rows[b],
                                      gsem[b]).wait()
                pltpu.async_copy(rows[b], acc_sh.at[didx[b]], ssem[b],
                                 add=True)
            return carry

        lax.fori_loop(0, K // NBUF, group, 0)
        bl = (K - 1) % NBUF
        pltpu.make_async_copy(rows[bl], acc_sh.at[didx[bl]], ssem[bl]).wait()
        plsc.subcore_barrier()
        pltpu.sync_copy(acc_sh.at[pl.ds(s * RPT, RPT)],
                        out_hbm.at[c, pl.ds(s * RPT, RPT)])

    return agg


def _sc_agg(hp, src3, dst3, zrows):
    return _make_sc_agg(D)(hp, src3, dst3, zrows)


def _sc_agg16(hp, src3, dst3, zrows):
    return _make_sc_agg(16)(hp, src3, dst3, zrows)


def _ln(xv, g, b):
    mu = jnp.mean(xv, axis=1, keepdims=True)
    xc = xv - mu
    var = jnp.mean(xc * xc, axis=1, keepdims=True)
    return xc * lax.rsqrt(var + 1e-5) * g + b


_AGG_SPEC = pl.BlockSpec((NC, BM, D), lambda i: (0, i, 0))
_ROW_SPEC = pl.BlockSpec((BM, D), lambda i: (i, 0))
_FULL_SPEC = pl.BlockSpec((D, D), lambda i: (0, 0))
_VEC_SPEC = pl.BlockSpec((1, D), lambda i: (0, 0))
_GRID = N // BM


_AGG16_SPEC = pl.BlockSpec((NC, BM, 16), lambda i: (0, i, 0))
_R16_SPEC = pl.BlockSpec((BM, 16), lambda i: (i, 0))


def _tc_dinv(deg16acc):
    def body(deg_ref, dinvf_ref, dinv16_ref):
        deg = deg_ref[0] + deg_ref[1]
        dinv = lax.rsqrt(jnp.maximum(deg, 1.0))
        dinv16_ref[...] = dinv
        dinvf_ref[...] = jnp.broadcast_to(dinv[:, 0:1], (BM, D))

    return pl.pallas_call(
        body,
        grid=(_GRID,),
        in_specs=[_AGG16_SPEC],
        out_specs=[_ROW_SPEC, _R16_SPEC],
        out_shape=[jax.ShapeDtypeStruct((N, D), jnp.float32),
                   jax.ShapeDtypeStruct((N, 16), jnp.float32)],
    )(deg16acc)


def _tc_head(s1acc, dinvf, bias, w0, w1):
    def body(s_ref, dinv_ref, b_ref, w0_ref, w1_ref, hp_ref):
        s1 = (s_ref[0] + s_ref[1])[:, 0:1]
        r = jnp.sum(w0_ref[...], axis=0, keepdims=True)
        dinv = dinv_ref[...]
        x1 = jnp.maximum(dinv * s1 * r + b_ref[...], 0.0)
        hp_ref[...] = dinv * jnp.dot(x1, w1_ref[...],
                                     preferred_element_type=jnp.float32)

    return pl.pallas_call(
        body,
        grid=(_GRID,),
        in_specs=[_AGG16_SPEC, _ROW_SPEC, _VEC_SPEC, _FULL_SPEC, _FULL_SPEC],
        out_specs=_ROW_SPEC,
        out_shape=jax.ShapeDtypeStruct((N, D), jnp.float32),
    )(s1acc, dinvf, bias, w0, w1)


def _tc_mid(aggout, dinvf, bias, wnext, z=None, add_one=False, ln=None,
            emit_x=False):
    ins = [aggout, dinvf, bias, wnext]
    specs = [_AGG_SPEC, _ROW_SPEC, _VEC_SPEC, _FULL_SPEC]
    if z is not None:
        ins.append(z)
        specs.append(_ROW_SPEC)
    if ln is not None:
        ins.extend(ln)
        specs.extend([_VEC_SPEC, _VEC_SPEC])

    def body(*refs):
        agg_ref, dinv_ref, b_ref, w_ref = refs[:4]
        pos = 4
        z_ref = None
        if z is not None:
            z_ref = refs[pos]
            pos += 1
        g_ref = bl_ref = None
        if ln is not None:
            g_ref, bl_ref = refs[pos], refs[pos + 1]
            pos += 2
        outs = refs[pos:]
        dinv = dinv_ref[...]
        xv = (agg_ref[0] + agg_ref[1]) * dinv + b_ref[...]
        if add_one:
            xv = xv + 1.0
        if z_ref is not None:
            xv = xv + z_ref[...]
        xv = jnp.maximum(xv, 0.0)
        if ln is not None:
            xv = _ln(xv, g_ref[...], bl_ref[...])
        outs[0][...] = dinv * jnp.dot(xv, w_ref[...],
                                      preferred_element_type=jnp.float32)
        if emit_x:
            outs[1][...] = xv

    n_out = 2 if emit_x else 1
    out = pl.pallas_call(
        body,
        grid=(_GRID,),
        in_specs=specs,
        out_specs=[_ROW_SPEC] * n_out,
        out_shape=[jax.ShapeDtypeStruct((N, D), jnp.float32)] * n_out,
    )(*ins)
    return out if emit_x else out[0]


def _tc_tail(aggout, dinvf, bias, z, g, b):
    def body(agg_ref, dinv_ref, b_ref, z_ref, g_ref, bl_ref, o_ref):
        xv = (agg_ref[0] + agg_ref[1]) * dinv_ref[...] + b_ref[...] + z_ref[...]
        o_ref[...] = _ln(xv, g_ref[...], bl_ref[...])

    return pl.pallas_call(
        body,
        grid=(_GRID,),
        in_specs=[_AGG_SPEC, _ROW_SPEC, _VEC_SPEC, _ROW_SPEC, _VEC_SPEC,
                  _VEC_SPEC],
        out_specs=_ROW_SPEC,
        out_shape=jax.ShapeDtypeStruct((N, D), jnp.float32),
    )(aggout, dinvf, bias, z, g, b)


@jax.jit
def kernel(x, edge_index, batch, Ws, bs, ln_hid_g, ln_hid_b, ln_out_g,
           ln_out_b):
    del x, batch
    loop = jnp.arange(N, dtype=jnp.int32)
    pad = E_PAD - E_RAW
    src = jnp.concatenate([edge_index[0].astype(jnp.int32), loop,
                           jnp.zeros((pad,), jnp.int32)])
    dst = jnp.concatenate([edge_index[1].astype(jnp.int32), loop,
                           jnp.full((pad,), N, jnp.int32)])
    src3 = src.reshape(NW, K, CH)
    dst3 = dst.reshape(NW, K, CH)
    zrows = jnp.zeros((NPAD, D), jnp.float32)
    zrows16 = jnp.zeros((NPAD, 16), jnp.float32)
    ones16 = jnp.ones((N, 16), jnp.float32)

    b2d = bs.reshape(9, 1, D)
    g_h = ln_hid_g.reshape(1, D)
    b_h = ln_hid_b.reshape(1, D)
    g_o = ln_out_g.reshape(1, D)
    b_o = ln_out_b.reshape(1, D)

    deg16 = _sc_agg16(ones16, src3, dst3, zrows16)
    dinvf, dinv16 = _tc_dinv(deg16)
    s1acc = _sc_agg16(dinv16, src3, dst3, zrows16)
    hp = _tc_head(s1acc, dinvf, b2d[0], Ws[0], Ws[1])

    z1 = z2 = None
    out = None
    for i in range(1, 9):
        agg = _sc_agg(hp, src3, dst3, zrows)
        if i == 2:
            hp, z1 = _tc_mid(agg, dinvf, b2d[2], Ws[3], add_one=True,
                             ln=(g_h, b_h), emit_x=True)
        elif i == 5:
            hp, z2 = _tc_mid(agg, dinvf, b2d[5], Ws[6], z=z1,
                             ln=(g_h, b_h), emit_x=True)
        elif i == 8:
            out = _tc_tail(agg, dinvf, b2d[8], z2, g_o, b_o)
        else:
            hp = _tc_mid(agg, dinvf, b2d[i], Ws[i + 1])
    return out
